# baseline (speedup 1.0000x reference)
import functools

import jax
import jax.numpy as jnp
import numpy as np
from jax import lax
from jax.experimental import pallas as pl
from jax.experimental.pallas import tpu as pltpu
from jax.experimental.pallas import tpu_sc as plsc

_BIAS = 500.0
_SIGMA = 20.0
_GAIN = 300.0
_NORM = 2000.0
_NBINS = 30000
_XG0 = 380.0
_XGL = 65535.0
_INV_DX = np.float32((_NBINS - 1) / (_XGL - _XG0))

_NC, _NS = 2, 16
_NW = _NC * _NS
_B, _H, _W = 16, 512, 512
_TOTAL = _B * _H * _W
_PIX = _H * _W
_CH = 1024

_NSPLIT = 2
_BS = _B // _NSPLIT
_TOT_S = _TOTAL // _NSPLIT
_PER_W = _TOT_S // _NW
_NPAIR = _PER_W // (2 * _CH)


def _sc_body(x_hbm, cl_hbm, ac_hbm, bc_hbm, ap_hbm, bp_hbm,
             u_hbm, p_hbm, v_hbm, ac_v, bc_v, ap_v, bp_v,
             xb0, cb0, ub0, pb0, vb0, xb1, cb1, ub1, pb1, vb1,
             sin0, sin1, sout0, sout1):
    wid = lax.axis_index("s") * _NC + lax.axis_index("c")
    base = wid * _PER_W
    tdesc = [pltpu.async_copy(src, dst, sin0) for src, dst in
             ((ac_hbm, ac_v), (bc_hbm, bc_v), (ap_hbm, ap_v), (bp_hbm, bp_v))]
    for d in tdesc:
        d.wait()

    def issue_in(ci, xb, cb, sem):
        off = base + ci * _CH
        pltpu.async_copy(x_hbm.at[pl.ds(off, _CH)], xb, sem)
        pltpu.async_copy(cl_hbm.at[pl.ds(off, _CH)], cb, sem)

    def drain_in(xb, cb, sem):
        pltpu.make_async_copy(x_hbm.at[pl.ds(base, _CH)], xb, sem).wait()
        pltpu.make_async_copy(cl_hbm.at[pl.ds(base, _CH)], cb, sem).wait()

    def issue_out(ci, ub, pb, vb, sem):
        off = base + ci * _CH
        pltpu.async_copy(ub, u_hbm.at[pl.ds(off, _CH)], sem)
        pltpu.async_copy(pb, p_hbm.at[pl.ds(off, _CH)], sem)
        pltpu.async_copy(vb, v_hbm.at[pl.ds(off, _CH)], sem)

    def drain_out(ub, pb, vb, sem):
        pltpu.make_async_copy(ub, u_hbm.at[pl.ds(base, _CH)], sem).wait()
        pltpu.make_async_copy(pb, p_hbm.at[pl.ds(base, _CH)], sem).wait()
        pltpu.make_async_copy(vb, v_hbm.at[pl.ds(base, _CH)], sem).wait()

    def compute(xb, cb, ub, pb, vb):
        @plsc.parallel_loop(0, _CH, step=16, unroll=4)
        def _(e):
            sl = pl.ds(e, 16)
            xv = xb[sl]
            cv = cb[sl]
            sig = jnp.maximum(cv * _NORM - _BIAS, 0.0)
            var = (2.0 * _GAIN) * sig + (_SIGMA * _SIGMA)
            vb[sl] = var
            r = lax.bitcast_convert_type(
                jnp.int32(0x5F3759DF) - lax.shift_right_arithmetic(
                    lax.bitcast_convert_type(var, jnp.int32), 1), jnp.float32)
            h = 0.5 * var
            r = r * (1.5 - h * r * r)
            r = r * (1.5 - h * r * r)
            sf = _SIGMA * r
            xc = (xv * _NORM) * sf + _BIAS
            xc = jnp.minimum(jnp.maximum(xc, _XG0), _XGL)
            posi = ((xc - _XG0) * _INV_DX).astype(jnp.int32)
            idx = jnp.maximum(jnp.minimum(posi + 1, _NBINS - 1), 1)
            ub[sl] = plsc.load_gather(ac_v, [idx]) + plsc.load_gather(bc_v, [idx]) * xc
            pb[sl] = plsc.load_gather(ap_v, [idx]) + plsc.load_gather(bp_v, [idx]) * xc

    issue_in(0, xb0, cb0, sin0)

    def pair(k, _):
        c0 = 2 * k
        issue_in(c0 + 1, xb1, cb1, sin1)
        drain_in(xb0, cb0, sin0)

        @pl.when(k > 0)
        def _():
            drain_out(ub0, pb0, vb0, sout0)

        compute(xb0, cb0, ub0, pb0, vb0)
        issue_out(c0, ub0, pb0, vb0, sout0)

        @pl.when(k < _NPAIR - 1)
        def _():
            issue_in(c0 + 2, xb0, cb0, sin0)

        drain_in(xb1, cb1, sin1)

        @pl.when(k > 0)
        def _():
            drain_out(ub1, pb1, vb1, sout1)

        compute(xb1, cb1, ub1, pb1, vb1)
        issue_out(c0 + 1, ub1, pb1, vb1, sout1)
        return _

    lax.fori_loop(0, _NPAIR, pair, None)
    drain_out(ub0, pb0, vb0, sout0)
    drain_out(ub1, pb1, vb1, sout1)


_sc_interp = functools.partial(
    pl.kernel,
    out_type=(jax.ShapeDtypeStruct((_TOT_S,), jnp.float32),
              jax.ShapeDtypeStruct((_TOT_S,), jnp.float32),
              jax.ShapeDtypeStruct((_TOT_S,), jnp.float32)),
    mesh=plsc.VectorSubcoreMesh(core_axis_name="c", subcore_axis_name="s",
                                num_cores=_NC, num_subcores=_NS),
    compiler_params=pltpu.CompilerParams(needs_layout_passes=False),
    scratch_types=(
        [pltpu.VMEM((_NBINS,), jnp.float32)] * 4
        + [pltpu.VMEM((_CH,), jnp.float32)] * 10
        + [pltpu.SemaphoreType.DMA] * 4
    ),
)(_sc_body)


_ROWS = 256
_STEPS_PER_IMG = _PIX // (128 * _ROWS)
_SLAB = 8

_SQRT2 = np.float32(np.sqrt(2.0))
_LD_CONST = np.float32(0.5 * np.log(2.0 * np.pi) + np.log(_NORM + 1e-8)
                       + np.log(_SIGMA))


def _erfinv(x):
    w = -jnp.log1p(-x * x)
    wc = w - 2.5
    p1 = jnp.float32(2.81022636e-08)
    for c in (3.43273939e-07, -3.5233877e-06, -4.39150654e-06, 0.00021858087,
              -0.00125372503, -0.00417768164, 0.246640727, 1.50140941):
        p1 = p1 * wc + jnp.float32(c)
    wt = jnp.sqrt(w) - 3.0
    p2 = jnp.float32(-0.000200214257)
    for c in (0.000100950558, 0.00134934322, -0.00367342844, 0.00573950773,
              -0.0076224613, 0.00943887047, 1.00167406, 2.83297682):
        p2 = p2 * wt + jnp.float32(c)
    return jnp.where(w < 5.0, p1, p2) * x


def _tc_body(u_ref, p_ref, v_ref, z_ref, ld_ref):
    j = pl.program_id(0) % _STEPS_PER_IMG

    @pl.when(j == 0)
    def _():
        ld_ref[...] = jnp.zeros((1, 1, 1), jnp.float32)

    def slab(i, acc):
        sl = (0, pl.ds(i * _SLAB, _SLAB), slice(None))
        u = jnp.clip(u_ref[sl], 1e-5, 1.0 - 1e-5)
        z = _erfinv(2.0 * u - 1.0) * _SQRT2
        z_ref[sl] = z
        return acc + (jnp.log(p_ref[sl] + 1e-8) + 0.5 * (z * z)
                      - 0.5 * jnp.log(v_ref[sl]))

    acc = lax.fori_loop(0, _ROWS // _SLAB, slab,
                        jnp.zeros((_SLAB, 128), jnp.float32), unroll=8)
    tot = jnp.sum(acc) + np.float32(_ROWS * 128) * _LD_CONST
    ld_ref[...] = ld_ref[...] + tot.reshape(1, 1, 1)


def _tc_finish(u, p, v):
    nsteps = _BS * _STEPS_PER_IMG
    return pl.pallas_call(
        _tc_body,
        grid=(nsteps,),
        in_specs=[pl.BlockSpec((1, _ROWS, 128),
                               lambda i: (i // _STEPS_PER_IMG,
                                          i % _STEPS_PER_IMG, 0))] * 3,
        out_specs=[pl.BlockSpec((1, _ROWS, 128),
                                lambda i: (i // _STEPS_PER_IMG,
                                           i % _STEPS_PER_IMG, 0)),
                   pl.BlockSpec((1, 1, 1),
                                lambda i: (i // _STEPS_PER_IMG, 0, 0))],
        out_shape=[jax.ShapeDtypeStruct((_BS, _PIX // 128, 128), jnp.float32),
                   jax.ShapeDtypeStruct((_BS, 1, 1), jnp.float32)],
    )(u, p, v)


def kernel(x, clean, x_grid, pdf_table, cdf_table):
    denom = (x_grid[1:] - x_grid[:-1]) + 1e-8
    b_c = (cdf_table[1:] - cdf_table[:-1]) / denom
    a_c = cdf_table[:-1] - b_c * x_grid[:-1]
    b_p = (pdf_table[1:] - pdf_table[:-1]) / denom
    a_p = pdf_table[:-1] - b_p * x_grid[:-1]
    pad = jnp.zeros((1,), jnp.float32)
    a_c = jnp.concatenate([pad, a_c])
    b_c = jnp.concatenate([pad, b_c])
    a_p = jnp.concatenate([pad, a_p])
    b_p = jnp.concatenate([pad, b_p])

    xh = x.reshape(_NSPLIT, _TOT_S)
    ch = clean.reshape(_NSPLIT, _TOT_S)
    s3 = (_BS, _PIX // 128, 128)
    scs = [_sc_interp(xh[h], ch[h], a_c, b_c, a_p, b_p)
           for h in range(_NSPLIT)]
    zs, lds = [], []
    for h in range(_NSPLIT):
        u, p, v = scs[h]
        z2, ld = _tc_finish(u.reshape(s3), p.reshape(s3), v.reshape(s3))
        zs.append(z2)
        lds.append(ld)
    z = jnp.concatenate(zs, axis=0).reshape(_B, 1, _H, _W)
    ld = jnp.concatenate(lds, axis=0).reshape(_B)
    return z, ld


# speedup vs baseline: 1.1672x; 1.1672x over previous
import functools

import jax
import jax.numpy as jnp
import numpy as np
from jax import lax
from jax.experimental import pallas as pl
from jax.experimental.pallas import tpu as pltpu
from jax.experimental.pallas import tpu_sc as plsc

_BIAS = 500.0
_SIGMA = 20.0
_GAIN = 300.0
_NORM = 2000.0
_NBINS = 30000
_XG0 = 380.0
_XGL = 65535.0
_INV_DX = np.float32((_NBINS - 1) / (_XGL - _XG0))

_NC, _NS = 2, 16
_NW = _NC * _NS
_B, _H, _W = 16, 512, 512
_TOTAL = _B * _H * _W
_PIX = _H * _W
_CH = 1024

_NSPLIT = 1
_BS = _B // _NSPLIT
_TOT_S = _TOTAL // _NSPLIT
_PER_W = _TOT_S // _NW
_NPAIR = _PER_W // (2 * _CH)


def _sc_body(x_hbm, cl_hbm, ac_hbm, bc_hbm, ap_hbm, bp_hbm,
             u_hbm, p_hbm, v_hbm, ac_v, bc_v, ap_v, bp_v,
             xb0, cb0, ub0, pb0, vb0, xb1, cb1, ub1, pb1, vb1,
             sin0, sin1, sout0, sout1):
    wid = lax.axis_index("s") * _NC + lax.axis_index("c")
    base = wid * _PER_W
    tdesc = [pltpu.async_copy(src, dst, sin0) for src, dst in
             ((ac_hbm, ac_v), (bc_hbm, bc_v), (ap_hbm, ap_v), (bp_hbm, bp_v))]
    for d in tdesc:
        d.wait()

    def issue_in(ci, xb, cb, sem):
        off = base + ci * _CH
        pltpu.async_copy(x_hbm.at[pl.ds(off, _CH)], xb, sem)
        pltpu.async_copy(cl_hbm.at[pl.ds(off, _CH)], cb, sem)

    def drain_in(xb, cb, sem):
        pltpu.make_async_copy(x_hbm.at[pl.ds(base, _CH)], xb, sem).wait()
        pltpu.make_async_copy(cl_hbm.at[pl.ds(base, _CH)], cb, sem).wait()

    def issue_out(ci, ub, pb, vb, sem):
        off = base + ci * _CH
        pltpu.async_copy(ub, u_hbm.at[pl.ds(off, _CH)], sem)
        pltpu.async_copy(pb, p_hbm.at[pl.ds(off, _CH)], sem)
        pltpu.async_copy(vb, v_hbm.at[pl.ds(off, _CH)], sem)

    def drain_out(ub, pb, vb, sem):
        pltpu.make_async_copy(ub, u_hbm.at[pl.ds(base, _CH)], sem).wait()
        pltpu.make_async_copy(pb, p_hbm.at[pl.ds(base, _CH)], sem).wait()
        pltpu.make_async_copy(vb, v_hbm.at[pl.ds(base, _CH)], sem).wait()

    def compute(xb, cb, ub, pb, vb):
        @plsc.parallel_loop(0, _CH, step=16, unroll=4)
        def _(e):
            sl = pl.ds(e, 16)
            xv = xb[sl]
            cv = cb[sl]
            sig = jnp.maximum(cv * _NORM - _BIAS, 0.0)
            var = (2.0 * _GAIN) * sig + (_SIGMA * _SIGMA)
            vb[sl] = var
            r = lax.bitcast_convert_type(
                jnp.int32(0x5F3759DF) - lax.shift_right_arithmetic(
                    lax.bitcast_convert_type(var, jnp.int32), 1), jnp.float32)
            h = 0.5 * var
            r = r * (1.5 - h * r * r)
            r = r * (1.5 - h * r * r)
            sf = _SIGMA * r
            xc = (xv * _NORM) * sf + _BIAS
            xc = jnp.minimum(jnp.maximum(xc, _XG0), _XGL)
            posi = ((xc - _XG0) * _INV_DX).astype(jnp.int32)
            idx = jnp.maximum(jnp.minimum(posi + 1, _NBINS - 1), 1)
            ub[sl] = plsc.load_gather(ac_v, [idx]) + plsc.load_gather(bc_v, [idx]) * xc
            pb[sl] = plsc.load_gather(ap_v, [idx]) + plsc.load_gather(bp_v, [idx]) * xc

    issue_in(0, xb0, cb0, sin0)

    def pair(k, _):
        c0 = 2 * k
        issue_in(c0 + 1, xb1, cb1, sin1)
        drain_in(xb0, cb0, sin0)

        @pl.when(k > 0)
        def _():
            drain_out(ub0, pb0, vb0, sout0)

        compute(xb0, cb0, ub0, pb0, vb0)
        issue_out(c0, ub0, pb0, vb0, sout0)

        @pl.when(k < _NPAIR - 1)
        def _():
            issue_in(c0 + 2, xb0, cb0, sin0)

        drain_in(xb1, cb1, sin1)

        @pl.when(k > 0)
        def _():
            drain_out(ub1, pb1, vb1, sout1)

        compute(xb1, cb1, ub1, pb1, vb1)
        issue_out(c0 + 1, ub1, pb1, vb1, sout1)
        return _

    lax.fori_loop(0, _NPAIR, pair, None)
    drain_out(ub0, pb0, vb0, sout0)
    drain_out(ub1, pb1, vb1, sout1)


_sc_interp = functools.partial(
    pl.kernel,
    out_type=(jax.ShapeDtypeStruct((_TOT_S,), jnp.float32),
              jax.ShapeDtypeStruct((_TOT_S,), jnp.float32),
              jax.ShapeDtypeStruct((_TOT_S,), jnp.float32)),
    mesh=plsc.VectorSubcoreMesh(core_axis_name="c", subcore_axis_name="s",
                                num_cores=_NC, num_subcores=_NS),
    compiler_params=pltpu.CompilerParams(needs_layout_passes=False),
    scratch_types=(
        [pltpu.VMEM((_NBINS,), jnp.float32)] * 4
        + [pltpu.VMEM((_CH,), jnp.float32)] * 10
        + [pltpu.SemaphoreType.DMA] * 4
    ),
)(_sc_body)


_ROWS = 256
_STEPS_PER_IMG = _PIX // (128 * _ROWS)
_SLAB = 8

_SQRT2 = np.float32(np.sqrt(2.0))
_LD_CONST = np.float32(0.5 * np.log(2.0 * np.pi) + np.log(_NORM + 1e-8)
                       + np.log(_SIGMA))


def _erfinv(x):
    w = -jnp.log1p(-x * x)
    wc = w - 2.5
    p1 = jnp.float32(2.81022636e-08)
    for c in (3.43273939e-07, -3.5233877e-06, -4.39150654e-06, 0.00021858087,
              -0.00125372503, -0.00417768164, 0.246640727, 1.50140941):
        p1 = p1 * wc + jnp.float32(c)
    wt = jnp.sqrt(w) - 3.0
    p2 = jnp.float32(-0.000200214257)
    for c in (0.000100950558, 0.00134934322, -0.00367342844, 0.00573950773,
              -0.0076224613, 0.00943887047, 1.00167406, 2.83297682):
        p2 = p2 * wt + jnp.float32(c)
    return jnp.where(w < 5.0, p1, p2) * x


def _tc_body(u_ref, p_ref, v_ref, z_ref, ld_ref):
    j = pl.program_id(0) % _STEPS_PER_IMG

    @pl.when(j == 0)
    def _():
        ld_ref[...] = jnp.zeros((1, 1, 1), jnp.float32)

    def slab(i, acc):
        sl = (0, pl.ds(i * _SLAB, _SLAB), slice(None))
        u = jnp.clip(u_ref[sl], 1e-5, 1.0 - 1e-5)
        z = _erfinv(2.0 * u - 1.0) * _SQRT2
        z_ref[sl] = z
        return acc + (jnp.log(p_ref[sl] + 1e-8) + 0.5 * (z * z)
                      - 0.5 * jnp.log(v_ref[sl]))

    acc = lax.fori_loop(0, _ROWS // _SLAB, slab,
                        jnp.zeros((_SLAB, 128), jnp.float32), unroll=8)
    tot = jnp.sum(acc) + np.float32(_ROWS * 128) * _LD_CONST
    ld_ref[...] = ld_ref[...] + tot.reshape(1, 1, 1)


def _tc_finish(u, p, v):
    nsteps = _BS * _STEPS_PER_IMG
    return pl.pallas_call(
        _tc_body,
        grid=(nsteps,),
        in_specs=[pl.BlockSpec((1, _ROWS, 128),
                               lambda i: (i // _STEPS_PER_IMG,
                                          i % _STEPS_PER_IMG, 0))] * 3,
        out_specs=[pl.BlockSpec((1, _ROWS, 128),
                                lambda i: (i // _STEPS_PER_IMG,
                                           i % _STEPS_PER_IMG, 0)),
                   pl.BlockSpec((1, 1, 1),
                                lambda i: (i // _STEPS_PER_IMG, 0, 0))],
        out_shape=[jax.ShapeDtypeStruct((_BS, _PIX // 128, 128), jnp.float32),
                   jax.ShapeDtypeStruct((_BS, 1, 1), jnp.float32)],
    )(u, p, v)


def kernel(x, clean, x_grid, pdf_table, cdf_table):
    denom = (x_grid[1:] - x_grid[:-1]) + 1e-8
    b_c = (cdf_table[1:] - cdf_table[:-1]) / denom
    a_c = cdf_table[:-1] - b_c * x_grid[:-1]
    b_p = (pdf_table[1:] - pdf_table[:-1]) / denom
    a_p = pdf_table[:-1] - b_p * x_grid[:-1]
    pad = jnp.zeros((1,), jnp.float32)
    a_c = jnp.concatenate([pad, a_c])
    b_c = jnp.concatenate([pad, b_c])
    a_p = jnp.concatenate([pad, a_p])
    b_p = jnp.concatenate([pad, b_p])

    s3 = (_BS, _PIX // 128, 128)
    u, p, v = _sc_interp(x.reshape(_TOT_S), clean.reshape(_TOT_S),
                         a_c, b_c, a_p, b_p)
    z2, ld = _tc_finish(u.reshape(s3), p.reshape(s3), v.reshape(s3))
    return z2.reshape(_B, 1, _H, _W), ld.reshape(_B)


# speedup vs baseline: 1.1991x; 1.0273x over previous
import functools

import jax
import jax.numpy as jnp
import numpy as np
from jax import lax
from jax.experimental import pallas as pl
from jax.experimental.pallas import tpu as pltpu
from jax.experimental.pallas import tpu_sc as plsc

_BIAS = 500.0
_SIGMA = 20.0
_GAIN = 300.0
_NORM = 2000.0
_NBINS = 30000
_XG0 = 380.0
_XGL = 65535.0
_INV_DX = np.float32((_NBINS - 1) / (_XGL - _XG0))

_NC, _NS = 2, 16
_NW = _NC * _NS
_B, _H, _W = 16, 512, 512
_TOTAL = _B * _H * _W
_PIX = _H * _W
_CH = 1024

_NSPLIT = 1
_BS = _B // _NSPLIT
_TOT_S = _TOTAL // _NSPLIT
_PER_W = _TOT_S // _NW
_NPAIR = _PER_W // (2 * _CH)


def _sc_body(x_hbm, cl_hbm, ac_hbm, bc_hbm, ap_hbm, bp_hbm,
             u_hbm, p_hbm, ac_v, bc_v, ap_v, bp_v,
             xb0, cb0, ub0, pb0, xb1, cb1, ub1, pb1,
             sin0, sin1, sout0, sout1):
    wid = lax.axis_index("s") * _NC + lax.axis_index("c")
    base = wid * _PER_W
    tdesc = [pltpu.async_copy(src, dst, sin0) for src, dst in
             ((ac_hbm, ac_v), (bc_hbm, bc_v), (ap_hbm, ap_v), (bp_hbm, bp_v))]
    for d in tdesc:
        d.wait()

    def issue_in(ci, xb, cb, sem):
        off = base + ci * _CH
        pltpu.async_copy(x_hbm.at[pl.ds(off, _CH)], xb, sem)
        pltpu.async_copy(cl_hbm.at[pl.ds(off, _CH)], cb, sem)

    def drain_in(xb, cb, sem):
        pltpu.make_async_copy(x_hbm.at[pl.ds(base, _CH)], xb, sem).wait()
        pltpu.make_async_copy(cl_hbm.at[pl.ds(base, _CH)], cb, sem).wait()

    def issue_out(ci, ub, pb, sem):
        off = base + ci * _CH
        pltpu.async_copy(ub, u_hbm.at[pl.ds(off, _CH)], sem)
        pltpu.async_copy(pb, p_hbm.at[pl.ds(off, _CH)], sem)

    def drain_out(ub, pb, sem):
        pltpu.make_async_copy(ub, u_hbm.at[pl.ds(base, _CH)], sem).wait()
        pltpu.make_async_copy(pb, p_hbm.at[pl.ds(base, _CH)], sem).wait()

    def compute(xb, cb, ub, pb):
        @plsc.parallel_loop(0, _CH, step=16, unroll=4)
        def _(e):
            sl = pl.ds(e, 16)
            xv = xb[sl]
            cv = cb[sl]
            sig = jnp.maximum(cv * _NORM - _BIAS, 0.0)
            var = (2.0 * _GAIN) * sig + (_SIGMA * _SIGMA)
            r = lax.bitcast_convert_type(
                jnp.int32(0x5F3759DF) - lax.shift_right_arithmetic(
                    lax.bitcast_convert_type(var, jnp.int32), 1), jnp.float32)
            h = 0.5 * var
            r = r * (1.5 - h * r * r)
            r = r * (1.5 - h * r * r)
            sf = _SIGMA * r
            xc = (xv * _NORM) * sf + _BIAS
            xc = jnp.minimum(jnp.maximum(xc, _XG0), _XGL)
            posi = ((xc - _XG0) * _INV_DX).astype(jnp.int32)
            idx = jnp.maximum(jnp.minimum(posi + 1, _NBINS - 1), 1)
            ub[sl] = plsc.load_gather(ac_v, [idx]) + plsc.load_gather(bc_v, [idx]) * xc
            p = plsc.load_gather(ap_v, [idx]) + plsc.load_gather(bp_v, [idx]) * xc
            pb[sl] = (p + 1e-8) * r

    issue_in(0, xb0, cb0, sin0)

    def pair(k, _):
        c0 = 2 * k
        issue_in(c0 + 1, xb1, cb1, sin1)
        drain_in(xb0, cb0, sin0)

        @pl.when(k > 0)
        def _():
            drain_out(ub0, pb0, sout0)

        compute(xb0, cb0, ub0, pb0)
        issue_out(c0, ub0, pb0, sout0)

        @pl.when(k < _NPAIR - 1)
        def _():
            issue_in(c0 + 2, xb0, cb0, sin0)

        drain_in(xb1, cb1, sin1)

        @pl.when(k > 0)
        def _():
            drain_out(ub1, pb1, sout1)

        compute(xb1, cb1, ub1, pb1)
        issue_out(c0 + 1, ub1, pb1, sout1)
        return _

    lax.fori_loop(0, _NPAIR, pair, None)
    drain_out(ub0, pb0, sout0)
    drain_out(ub1, pb1, sout1)


_sc_interp = functools.partial(
    pl.kernel,
    out_type=(jax.ShapeDtypeStruct((_TOT_S,), jnp.float32),
              jax.ShapeDtypeStruct((_TOT_S,), jnp.float32)),
    mesh=plsc.VectorSubcoreMesh(core_axis_name="c", subcore_axis_name="s",
                                num_cores=_NC, num_subcores=_NS),
    compiler_params=pltpu.CompilerParams(needs_layout_passes=False),
    scratch_types=(
        [pltpu.VMEM((_NBINS,), jnp.float32)] * 4
        + [pltpu.VMEM((_CH,), jnp.float32)] * 8
        + [pltpu.SemaphoreType.DMA] * 4
    ),
)(_sc_body)


_ROWS = 256
_STEPS_PER_IMG = _PIX // (128 * _ROWS)
_SLAB = 8

_SQRT2 = np.float32(np.sqrt(2.0))
_LD_CONST = np.float32(0.5 * np.log(2.0 * np.pi) + np.log(_NORM + 1e-8)
                       + np.log(_SIGMA))


def _erfinv(x):
    w = -jnp.log1p(-x * x)
    wc = w - 2.5
    p1 = jnp.float32(2.81022636e-08)
    for c in (3.43273939e-07, -3.5233877e-06, -4.39150654e-06, 0.00021858087,
              -0.00125372503, -0.00417768164, 0.246640727, 1.50140941):
        p1 = p1 * wc + jnp.float32(c)
    wt = jnp.sqrt(w) - 3.0
    p2 = jnp.float32(-0.000200214257)
    for c in (0.000100950558, 0.00134934322, -0.00367342844, 0.00573950773,
              -0.0076224613, 0.00943887047, 1.00167406, 2.83297682):
        p2 = p2 * wt + jnp.float32(c)
    return jnp.where(w < 5.0, p1, p2) * x


def _tc_body(u_ref, q_ref, z_ref, ld_ref):
    j = pl.program_id(0) % _STEPS_PER_IMG

    @pl.when(j == 0)
    def _():
        ld_ref[...] = jnp.zeros((1, 1, 1), jnp.float32)

    def slab(i, acc):
        sl = (0, pl.ds(i * _SLAB, _SLAB), slice(None))
        u = jnp.clip(u_ref[sl], 1e-5, 1.0 - 1e-5)
        z = _erfinv(2.0 * u - 1.0) * _SQRT2
        z_ref[sl] = z
        return acc + (jnp.log(q_ref[sl]) + 0.5 * (z * z))

    acc = lax.fori_loop(0, _ROWS // _SLAB, slab,
                        jnp.zeros((_SLAB, 128), jnp.float32), unroll=8)
    tot = jnp.sum(acc) + np.float32(_ROWS * 128) * _LD_CONST
    ld_ref[...] = ld_ref[...] + tot.reshape(1, 1, 1)


def _tc_finish(u, q):
    nsteps = _BS * _STEPS_PER_IMG
    return pl.pallas_call(
        _tc_body,
        grid=(nsteps,),
        in_specs=[pl.BlockSpec((1, _ROWS, 128),
                               lambda i: (i // _STEPS_PER_IMG,
                                          i % _STEPS_PER_IMG, 0))] * 2,
        out_specs=[pl.BlockSpec((1, _ROWS, 128),
                                lambda i: (i // _STEPS_PER_IMG,
                                           i % _STEPS_PER_IMG, 0)),
                   pl.BlockSpec((1, 1, 1),
                                lambda i: (i // _STEPS_PER_IMG, 0, 0))],
        out_shape=[jax.ShapeDtypeStruct((_BS, _PIX // 128, 128), jnp.float32),
                   jax.ShapeDtypeStruct((_BS, 1, 1), jnp.float32)],
    )(u, q)


def kernel(x, clean, x_grid, pdf_table, cdf_table):
    denom = (x_grid[1:] - x_grid[:-1]) + 1e-8
    b_c = (cdf_table[1:] - cdf_table[:-1]) / denom
    a_c = cdf_table[:-1] - b_c * x_grid[:-1]
    b_p = (pdf_table[1:] - pdf_table[:-1]) / denom
    a_p = pdf_table[:-1] - b_p * x_grid[:-1]
    pad = jnp.zeros((1,), jnp.float32)
    a_c = jnp.concatenate([pad, a_c])
    b_c = jnp.concatenate([pad, b_c])
    a_p = jnp.concatenate([pad, a_p])
    b_p = jnp.concatenate([pad, b_p])

    s3 = (_BS, _PIX // 128, 128)
    u, q = _sc_interp(x.reshape(_TOT_S), clean.reshape(_TOT_S),
                      a_c, b_c, a_p, b_p)
    z2, ld = _tc_finish(u.reshape(s3), q.reshape(s3))
    return z2.reshape(_B, 1, _H, _W), ld.reshape(_B)


# speedup vs baseline: 1.2818x; 1.0690x over previous
import functools

import jax
import jax.numpy as jnp
import numpy as np
from jax import lax
from jax.experimental import pallas as pl
from jax.experimental.pallas import tpu as pltpu
from jax.experimental.pallas import tpu_sc as plsc

_BIAS = 500.0
_SIGMA = 20.0
_GAIN = 300.0
_NORM = 2000.0
_NBINS = 30000
_XG0 = 380.0
_XGL = 65535.0
_INV_DX = np.float32((_NBINS - 1) / (_XGL - _XG0))

_NC, _NS = 2, 16
_NW = _NC * _NS
_B, _H, _W = 16, 512, 512
_TOTAL = _B * _H * _W
_PIX = _H * _W
_CH = 1024

_NSPLIT = 1
_BS = _B // _NSPLIT
_TOT_S = _TOTAL // _NSPLIT
_PER_W = _TOT_S // _NW
_NPAIR = _PER_W // (2 * _CH)


def _sc_body(x_hbm, cl_hbm, ac_hbm, bc_hbm, ap_hbm, bp_hbm,
             u_hbm, p_hbm, ac_v, bc_v, ap_v, bp_v,
             xb0, cb0, ub0, pb0, xb1, cb1, ub1, pb1,
             sin0, sin1, sout0, sout1):
    wid = lax.axis_index("s") * _NC + lax.axis_index("c")
    base = wid * _PER_W
    tdesc = [pltpu.async_copy(src, dst, sin0) for src, dst in
             ((ac_hbm, ac_v), (bc_hbm, bc_v), (ap_hbm, ap_v), (bp_hbm, bp_v))]
    for d in tdesc:
        d.wait()

    def issue_in(ci, xb, cb, sem):
        off = base + ci * _CH
        pltpu.async_copy(x_hbm.at[pl.ds(off, _CH)], xb, sem)
        pltpu.async_copy(cl_hbm.at[pl.ds(off, _CH)], cb, sem)

    def drain_in(xb, cb, sem):
        pltpu.make_async_copy(x_hbm.at[pl.ds(base, _CH)], xb, sem).wait()
        pltpu.make_async_copy(cl_hbm.at[pl.ds(base, _CH)], cb, sem).wait()

    def issue_out(ci, ub, pb, sem):
        off = base + ci * _CH
        pltpu.async_copy(ub, u_hbm.at[pl.ds(off, _CH)], sem)
        pltpu.async_copy(pb, p_hbm.at[pl.ds(off, _CH)], sem)

    def drain_out(ub, pb, sem):
        pltpu.make_async_copy(ub, u_hbm.at[pl.ds(base, _CH)], sem).wait()
        pltpu.make_async_copy(pb, p_hbm.at[pl.ds(base, _CH)], sem).wait()

    def compute(xb, cb, ub, pb):
        @plsc.parallel_loop(0, _CH, step=16, unroll=4)
        def _(e):
            sl = pl.ds(e, 16)
            xv = xb[sl]
            cv = cb[sl]
            sig = jnp.maximum(cv * _NORM - _BIAS, 0.0)
            var = (2.0 * _GAIN) * sig + (_SIGMA * _SIGMA)
            r = lax.bitcast_convert_type(
                jnp.int32(0x5F3759DF) - lax.shift_right_arithmetic(
                    lax.bitcast_convert_type(var, jnp.int32), 1), jnp.float32)
            h = 0.5 * var
            r = r * (1.5 - h * r * r)
            r = r * (1.5 - h * r * r)
            sf = _SIGMA * r
            xc = (xv * _NORM) * sf + _BIAS
            xc = jnp.minimum(jnp.maximum(xc, _XG0), _XGL)
            posi = ((xc - _XG0) * _INV_DX).astype(jnp.int32)
            idx = jnp.maximum(jnp.minimum(posi + 1, _NBINS - 1), 1)
            ub[sl] = plsc.load_gather(ac_v, [idx]) + plsc.load_gather(bc_v, [idx]) * xc
            p = plsc.load_gather(ap_v, [idx]) + plsc.load_gather(bp_v, [idx]) * xc
            pb[sl] = (p + 1e-8) * r

    issue_in(0, xb0, cb0, sin0)

    def pair(k, _):
        c0 = 2 * k
        issue_in(c0 + 1, xb1, cb1, sin1)
        drain_in(xb0, cb0, sin0)

        @pl.when(k > 0)
        def _():
            drain_out(ub0, pb0, sout0)

        compute(xb0, cb0, ub0, pb0)
        issue_out(c0, ub0, pb0, sout0)

        @pl.when(k < _NPAIR - 1)
        def _():
            issue_in(c0 + 2, xb0, cb0, sin0)

        drain_in(xb1, cb1, sin1)

        @pl.when(k > 0)
        def _():
            drain_out(ub1, pb1, sout1)

        compute(xb1, cb1, ub1, pb1)
        issue_out(c0 + 1, ub1, pb1, sout1)
        return _

    lax.fori_loop(0, _NPAIR, pair, None)
    drain_out(ub0, pb0, sout0)
    drain_out(ub1, pb1, sout1)


_sc_interp = functools.partial(
    pl.kernel,
    out_type=(jax.ShapeDtypeStruct((_TOT_S,), jnp.float32),
              jax.ShapeDtypeStruct((_TOT_S,), jnp.float32)),
    mesh=plsc.VectorSubcoreMesh(core_axis_name="c", subcore_axis_name="s",
                                num_cores=_NC, num_subcores=_NS),
    compiler_params=pltpu.CompilerParams(needs_layout_passes=False),
    scratch_types=(
        [pltpu.VMEM((_NBINS,), jnp.float32)] * 4
        + [pltpu.VMEM((_CH,), jnp.float32)] * 8
        + [pltpu.SemaphoreType.DMA] * 4
    ),
)(_sc_body)


_ROWS = 256
_STEPS_PER_IMG = _PIX // (128 * _ROWS)
_SLAB = 32

_SQRT2 = np.float32(np.sqrt(2.0))
_LD_CONST = np.float32(0.5 * np.log(2.0 * np.pi) + np.log(_NORM + 1e-8)
                       + np.log(_SIGMA))


def _erfinv(x):
    w = -jnp.log1p(-x * x)
    wc = w - 2.5
    p1 = jnp.float32(2.81022636e-08)
    for c in (3.43273939e-07, -3.5233877e-06, -4.39150654e-06, 0.00021858087,
              -0.00125372503, -0.00417768164, 0.246640727, 1.50140941):
        p1 = p1 * wc + jnp.float32(c)
    wt = jnp.sqrt(w) - 3.0
    p2 = jnp.float32(-0.000200214257)
    for c in (0.000100950558, 0.00134934322, -0.00367342844, 0.00573950773,
              -0.0076224613, 0.00943887047, 1.00167406, 2.83297682):
        p2 = p2 * wt + jnp.float32(c)
    return jnp.where(w < 5.0, p1, p2) * x


def _tc_body(u_ref, q_ref, z_ref, ld_ref):
    j = pl.program_id(0) % _STEPS_PER_IMG

    @pl.when(j == 0)
    def _():
        ld_ref[...] = jnp.zeros((1, 1, 1), jnp.float32)

    def slab(i, acc):
        sl = (0, pl.ds(i * _SLAB, _SLAB), slice(None))
        u = jnp.clip(u_ref[sl], 1e-5, 1.0 - 1e-5)
        z = _erfinv(2.0 * u - 1.0) * _SQRT2
        z_ref[0, 0, pl.ds(i * 8, 8), :] = z.reshape(8, _W)
        return acc + (jnp.log(q_ref[sl]) + 0.5 * (z * z))

    acc = lax.fori_loop(0, _ROWS // _SLAB, slab,
                        jnp.zeros((_SLAB, 128), jnp.float32), unroll=2)
    tot = jnp.sum(acc) + np.float32(_ROWS * 128) * _LD_CONST
    ld_ref[...] = ld_ref[...] + tot.reshape(1, 1, 1)


def _tc_finish(u, q):
    nsteps = _BS * _STEPS_PER_IMG
    return pl.pallas_call(
        _tc_body,
        grid=(nsteps,),
        in_specs=[pl.BlockSpec((1, _ROWS, 128),
                               lambda i: (i // _STEPS_PER_IMG,
                                          i % _STEPS_PER_IMG, 0))] * 2,
        out_specs=[pl.BlockSpec((1, 1, _H // _STEPS_PER_IMG, _W),
                                lambda i: (i // _STEPS_PER_IMG, 0,
                                           i % _STEPS_PER_IMG, 0)),
                   pl.BlockSpec((1, 1, 1),
                                lambda i: (i // _STEPS_PER_IMG, 0, 0))],
        out_shape=[jax.ShapeDtypeStruct((_BS, 1, _H, _W), jnp.float32),
                   jax.ShapeDtypeStruct((_BS, 1, 1), jnp.float32)],
    )(u, q)


def kernel(x, clean, x_grid, pdf_table, cdf_table):
    denom = (x_grid[1:] - x_grid[:-1]) + 1e-8
    b_c = (cdf_table[1:] - cdf_table[:-1]) / denom
    a_c = cdf_table[:-1] - b_c * x_grid[:-1]
    b_p = (pdf_table[1:] - pdf_table[:-1]) / denom
    a_p = pdf_table[:-1] - b_p * x_grid[:-1]
    pad = jnp.zeros((1,), jnp.float32)
    a_c = jnp.concatenate([pad, a_c])
    b_c = jnp.concatenate([pad, b_c])
    a_p = jnp.concatenate([pad, a_p])
    b_p = jnp.concatenate([pad, b_p])

    s3 = (_BS, _PIX // 128, 128)
    u, q = _sc_interp(x.reshape(_TOT_S), clean.reshape(_TOT_S),
                      a_c, b_c, a_p, b_p)
    z2, ld = _tc_finish(u.reshape(s3), q.reshape(s3))
    return z2, ld.reshape(_B)


# speedup vs baseline: 1.4644x; 1.1424x over previous
import functools

import jax
import jax.numpy as jnp
import numpy as np
from jax import lax
from jax.experimental import pallas as pl
from jax.experimental.pallas import tpu as pltpu
from jax.experimental.pallas import tpu_sc as plsc

_BIAS = 500.0
_SIGMA = 20.0
_GAIN = 300.0
_NORM = 2000.0
_NBINS = 30000
_XG0 = 380.0
_XGL = 65535.0
_INV_DX = np.float32((_NBINS - 1) / (_XGL - _XG0))

_NC, _NS = 2, 16
_NW = _NC * _NS
_B, _H, _W = 16, 512, 512
_TOTAL = _B * _H * _W
_PIX = _H * _W
_CH = 1024

_NSPLIT = 1
_BS = _B // _NSPLIT
_TOT_S = _TOTAL // _NSPLIT
_PER_W = _TOT_S // _NW
_NPAIR = _PER_W // (2 * _CH)


def _sc_body(x_hbm, cl_hbm, ac_hbm, bc_hbm, ap_hbm, bp_hbm,
             u_hbm, p_hbm, ac_v, bc_v, ap_v, bp_v,
             xb0, cb0, ub0, pb0, xb1, cb1, ub1, pb1,
             sin0, sin1, sout0, sout1):
    wid = lax.axis_index("s") * _NC + lax.axis_index("c")
    base = wid * _PER_W
    tdesc = [pltpu.async_copy(src, dst, sin0) for src, dst in
             ((ac_hbm, ac_v), (bc_hbm, bc_v), (ap_hbm, ap_v), (bp_hbm, bp_v))]
    for d in tdesc:
        d.wait()

    def issue_in(ci, xb, cb, sem):
        tci = base // _CH + ci
        img = lax.shift_right_logical(tci, 8)
        rem = lax.bitwise_and(tci, 255)
        t = lax.shift_right_logical(rem, 2)
        c = lax.bitwise_and(rem, 3)
        tile = (img, 0, pl.ds(8 * t, 8), pl.ds(128 * c, 128))
        pltpu.async_copy(x_hbm.at[tile], xb, sem)
        pltpu.async_copy(cl_hbm.at[tile], cb, sem)

    def drain_in(xb, cb, sem):
        tile0 = (0, 0, pl.ds(0, 8), pl.ds(0, 128))
        pltpu.make_async_copy(x_hbm.at[tile0], xb, sem).wait()
        pltpu.make_async_copy(cl_hbm.at[tile0], cb, sem).wait()

    def issue_out(ci, ub, pb, sem):
        off = base + ci * _CH
        pltpu.async_copy(ub, u_hbm.at[pl.ds(off, _CH)], sem)
        pltpu.async_copy(pb, p_hbm.at[pl.ds(off, _CH)], sem)

    def drain_out(ub, pb, sem):
        pltpu.make_async_copy(ub, u_hbm.at[pl.ds(base, _CH)], sem).wait()
        pltpu.make_async_copy(pb, p_hbm.at[pl.ds(base, _CH)], sem).wait()

    def compute(xb, cb, ub, pb):
        @plsc.parallel_loop(0, _CH // 16, step=1, unroll=4)
        def _(e):
            row = lax.shift_right_logical(e, 3)
            col = 16 * lax.bitwise_and(e, 7)
            sl = pl.ds(16 * e, 16)
            xv = xb[row, pl.ds(col, 16)]
            cv = cb[row, pl.ds(col, 16)]
            sig = jnp.maximum(cv * _NORM - _BIAS, 0.0)
            var = (2.0 * _GAIN) * sig + (_SIGMA * _SIGMA)
            r = lax.bitcast_convert_type(
                jnp.int32(0x5F3759DF) - lax.shift_right_arithmetic(
                    lax.bitcast_convert_type(var, jnp.int32), 1), jnp.float32)
            h = 0.5 * var
            r = r * (1.5 - h * r * r)
            r = r * (1.5 - h * r * r)
            sf = _SIGMA * r
            xc = (xv * _NORM) * sf + _BIAS
            xc = jnp.minimum(jnp.maximum(xc, _XG0), _XGL)
            posi = ((xc - _XG0) * _INV_DX).astype(jnp.int32)
            idx = jnp.maximum(jnp.minimum(posi + 1, _NBINS - 1), 1)
            ub[sl] = plsc.load_gather(ac_v, [idx]) + plsc.load_gather(bc_v, [idx]) * xc
            p = plsc.load_gather(ap_v, [idx]) + plsc.load_gather(bp_v, [idx]) * xc
            pb[sl] = (p + 1e-8) * r

    issue_in(0, xb0, cb0, sin0)

    def pair(k, _):
        c0 = 2 * k
        issue_in(c0 + 1, xb1, cb1, sin1)
        drain_in(xb0, cb0, sin0)

        @pl.when(k > 0)
        def _():
            drain_out(ub0, pb0, sout0)

        compute(xb0, cb0, ub0, pb0)
        issue_out(c0, ub0, pb0, sout0)

        @pl.when(k < _NPAIR - 1)
        def _():
            issue_in(c0 + 2, xb0, cb0, sin0)

        drain_in(xb1, cb1, sin1)

        @pl.when(k > 0)
        def _():
            drain_out(ub1, pb1, sout1)

        compute(xb1, cb1, ub1, pb1)
        issue_out(c0 + 1, ub1, pb1, sout1)
        return _

    lax.fori_loop(0, _NPAIR, pair, None)
    drain_out(ub0, pb0, sout0)
    drain_out(ub1, pb1, sout1)


_sc_interp = functools.partial(
    pl.kernel,
    out_type=(jax.ShapeDtypeStruct((_TOT_S,), jnp.float32),
              jax.ShapeDtypeStruct((_TOT_S,), jnp.float32)),
    mesh=plsc.VectorSubcoreMesh(core_axis_name="c", subcore_axis_name="s",
                                num_cores=_NC, num_subcores=_NS),
    compiler_params=pltpu.CompilerParams(needs_layout_passes=False,
                                         use_tc_tiling_on_sc=True),
    scratch_types=(
        [pltpu.VMEM((_NBINS,), jnp.float32)] * 4
        + [pltpu.VMEM((8, 128), jnp.float32),
           pltpu.VMEM((8, 128), jnp.float32),
           pltpu.VMEM((_CH,), jnp.float32),
           pltpu.VMEM((_CH,), jnp.float32)] * 2
        + [pltpu.SemaphoreType.DMA] * 4
    ),
)(_sc_body)


_ROWS = 256
_STEPS_PER_IMG = _PIX // (128 * _ROWS)
_SLAB = 32

_SQRT2 = np.float32(np.sqrt(2.0))
_LD_CONST = np.float32(0.5 * np.log(2.0 * np.pi) + np.log(_NORM + 1e-8)
                       + np.log(_SIGMA))


def _erfinv(x):
    w = -jnp.log1p(-x * x)
    wc = w - 2.5
    p1 = jnp.float32(2.81022636e-08)
    for c in (3.43273939e-07, -3.5233877e-06, -4.39150654e-06, 0.00021858087,
              -0.00125372503, -0.00417768164, 0.246640727, 1.50140941):
        p1 = p1 * wc + jnp.float32(c)
    wt = jnp.sqrt(w) - 3.0
    p2 = jnp.float32(-0.000200214257)
    for c in (0.000100950558, 0.00134934322, -0.00367342844, 0.00573950773,
              -0.0076224613, 0.00943887047, 1.00167406, 2.83297682):
        p2 = p2 * wt + jnp.float32(c)
    return jnp.where(w < 5.0, p1, p2) * x


def _tc_body(u_ref, q_ref, z_ref, ld_ref):
    j = pl.program_id(0) % _STEPS_PER_IMG

    @pl.when(j == 0)
    def _():
        ld_ref[...] = jnp.zeros((1, 1, 1), jnp.float32)

    def slab(i, acc):
        sl = (0, pl.ds(i * _SLAB, _SLAB), slice(None))
        u = jnp.clip(u_ref[sl], 1e-5, 1.0 - 1e-5)
        z = _erfinv(2.0 * u - 1.0) * _SQRT2
        z_ref[0, 0, pl.ds(i * 8, 8), :] = (
            z.reshape(4, 8, 128).transpose(1, 0, 2).reshape(8, _W))
        return acc + (jnp.log(q_ref[sl]) + 0.5 * (z * z))

    acc = lax.fori_loop(0, _ROWS // _SLAB, slab,
                        jnp.zeros((_SLAB, 128), jnp.float32), unroll=2)
    tot = jnp.sum(acc) + np.float32(_ROWS * 128) * _LD_CONST
    ld_ref[...] = ld_ref[...] + tot.reshape(1, 1, 1)


def _tc_finish(u, q):
    nsteps = _BS * _STEPS_PER_IMG
    return pl.pallas_call(
        _tc_body,
        grid=(nsteps,),
        in_specs=[pl.BlockSpec((1, _ROWS, 128),
                               lambda i: (i // _STEPS_PER_IMG,
                                          i % _STEPS_PER_IMG, 0))] * 2,
        out_specs=[pl.BlockSpec((1, 1, _H // _STEPS_PER_IMG, _W),
                                lambda i: (i // _STEPS_PER_IMG, 0,
                                           i % _STEPS_PER_IMG, 0)),
                   pl.BlockSpec((1, 1, 1),
                                lambda i: (i // _STEPS_PER_IMG, 0, 0))],
        out_shape=[jax.ShapeDtypeStruct((_BS, 1, _H, _W), jnp.float32),
                   jax.ShapeDtypeStruct((_BS, 1, 1), jnp.float32)],
    )(u, q)


def kernel(x, clean, x_grid, pdf_table, cdf_table):
    denom = (x_grid[1:] - x_grid[:-1]) + 1e-8
    b_c = (cdf_table[1:] - cdf_table[:-1]) / denom
    a_c = cdf_table[:-1] - b_c * x_grid[:-1]
    b_p = (pdf_table[1:] - pdf_table[:-1]) / denom
    a_p = pdf_table[:-1] - b_p * x_grid[:-1]
    pad = jnp.zeros((1,), jnp.float32)
    a_c = jnp.concatenate([pad, a_c])
    b_c = jnp.concatenate([pad, b_c])
    a_p = jnp.concatenate([pad, a_p])
    b_p = jnp.concatenate([pad, b_p])

    s3 = (_BS, _PIX // 128, 128)
    u, q = _sc_interp(x, clean, a_c, b_c, a_p, b_p)
    z2, ld = _tc_finish(u.reshape(s3), q.reshape(s3))
    return z2, ld.reshape(_B)


# speedup vs baseline: 1.6729x; 1.1424x over previous
import functools

import jax
import jax.numpy as jnp
import numpy as np
from jax import lax
from jax.experimental import pallas as pl
from jax.experimental.pallas import tpu as pltpu
from jax.experimental.pallas import tpu_sc as plsc

_BIAS = 500.0
_SIGMA = 20.0
_GAIN = 300.0
_NORM = 2000.0
_NBINS = 30000
_XG0 = 380.0
_XGL = 65535.0
_INV_DX = np.float32((_NBINS - 1) / (_XGL - _XG0))

_NC, _NS = 2, 16
_NW = _NC * _NS
_B, _H, _W = 16, 512, 512
_TOTAL = _B * _H * _W
_PIX = _H * _W
_CH = 1024

_NSPLIT = 1
_BS = _B // _NSPLIT
_TOT_S = _TOTAL // _NSPLIT
_PER_W = _TOT_S // _NW
_NPAIR = _PER_W // (2 * _CH)


def _sc_body(x_hbm, cl_hbm, ac_hbm, bc_hbm, ap_hbm, bp_hbm,
             u_hbm, p_hbm, ac_v, bc_v, ap_v, bp_v,
             xb0, cb0, ub0, pb0, xb1, cb1, ub1, pb1,
             sin0, sin1, sout0, sout1):
    wid = lax.axis_index("s") * _NC + lax.axis_index("c")
    base = wid * _PER_W
    tdesc = [pltpu.async_copy(src, dst, sin0) for src, dst in
             ((ac_hbm, ac_v), (bc_hbm, bc_v), (ap_hbm, ap_v), (bp_hbm, bp_v))]
    for d in tdesc:
        d.wait()

    def issue_in(ci, xb, cb, sem):
        tci = base // _CH + ci
        img = lax.shift_right_logical(tci, 8)
        rem = lax.bitwise_and(tci, 255)
        t = lax.shift_right_logical(rem, 2)
        c = lax.bitwise_and(rem, 3)
        tile = (img, 0, pl.ds(8 * t, 8), pl.ds(128 * c, 128))
        pltpu.async_copy(x_hbm.at[tile], xb, sem)
        pltpu.async_copy(cl_hbm.at[tile], cb, sem)

    def drain_in(xb, cb, sem):
        tile0 = (0, 0, pl.ds(0, 8), pl.ds(0, 128))
        pltpu.make_async_copy(x_hbm.at[tile0], xb, sem).wait()
        pltpu.make_async_copy(cl_hbm.at[tile0], cb, sem).wait()

    def issue_out(ci, ub, pb, sem):
        off = base + ci * _CH
        pltpu.async_copy(ub, u_hbm.at[pl.ds(off, _CH)], sem)
        pltpu.async_copy(pb, p_hbm.at[pl.ds(off, _CH)], sem)

    def drain_out(ub, pb, sem):
        pltpu.make_async_copy(ub, u_hbm.at[pl.ds(base, _CH)], sem).wait()
        pltpu.make_async_copy(pb, p_hbm.at[pl.ds(base, _CH)], sem).wait()

    def compute(xb, cb, ub, pb):
        @plsc.parallel_loop(0, _CH // 16, step=1, unroll=8)
        def _(e):
            row = lax.shift_right_logical(e, 3)
            col = 16 * lax.bitwise_and(e, 7)
            sl = pl.ds(16 * e, 16)
            xv = xb[row, pl.ds(col, 16)]
            cv = cb[row, pl.ds(col, 16)]
            sig = jnp.maximum(cv * _NORM - _BIAS, 0.0)
            var = (2.0 * _GAIN) * sig + (_SIGMA * _SIGMA)
            r = lax.bitcast_convert_type(
                jnp.int32(0x5F3759DF) - lax.shift_right_arithmetic(
                    lax.bitcast_convert_type(var, jnp.int32), 1), jnp.float32)
            h = 0.5 * var
            r = r * (1.5 - h * r * r)
            r = r * (1.5 - h * r * r)
            sf = _SIGMA * r
            xc = (xv * _NORM) * sf + _BIAS
            xc = jnp.minimum(jnp.maximum(xc, _XG0), _XGL)
            posi = ((xc - _XG0) * _INV_DX).astype(jnp.int32)
            idx = jnp.maximum(jnp.minimum(posi + 1, _NBINS - 1), 1)
            ub[sl] = plsc.load_gather(ac_v, [idx]) + plsc.load_gather(bc_v, [idx]) * xc
            p = plsc.load_gather(ap_v, [idx]) + plsc.load_gather(bp_v, [idx]) * xc
            pb[sl] = (p + 1e-8) * r

    issue_in(0, xb0, cb0, sin0)

    def pair(k, _):
        c0 = 2 * k
        issue_in(c0 + 1, xb1, cb1, sin1)
        drain_in(xb0, cb0, sin0)

        @pl.when(k > 0)
        def _():
            drain_out(ub0, pb0, sout0)

        compute(xb0, cb0, ub0, pb0)
        issue_out(c0, ub0, pb0, sout0)

        @pl.when(k < _NPAIR - 1)
        def _():
            issue_in(c0 + 2, xb0, cb0, sin0)

        drain_in(xb1, cb1, sin1)

        @pl.when(k > 0)
        def _():
            drain_out(ub1, pb1, sout1)

        compute(xb1, cb1, ub1, pb1)
        issue_out(c0 + 1, ub1, pb1, sout1)
        return _

    lax.fori_loop(0, _NPAIR, pair, None)
    drain_out(ub0, pb0, sout0)
    drain_out(ub1, pb1, sout1)


_sc_interp = functools.partial(
    pl.kernel,
    out_type=(jax.ShapeDtypeStruct((_TOT_S,), jnp.float32),
              jax.ShapeDtypeStruct((_TOT_S,), jnp.float32)),
    mesh=plsc.VectorSubcoreMesh(core_axis_name="c", subcore_axis_name="s",
                                num_cores=_NC, num_subcores=_NS),
    compiler_params=pltpu.CompilerParams(needs_layout_passes=False,
                                         use_tc_tiling_on_sc=True),
    scratch_types=(
        [pltpu.VMEM((_NBINS,), jnp.float32)] * 4
        + [pltpu.VMEM((8, 128), jnp.float32),
           pltpu.VMEM((8, 128), jnp.float32),
           pltpu.VMEM((_CH,), jnp.float32),
           pltpu.VMEM((_CH,), jnp.float32)] * 2
        + [pltpu.SemaphoreType.DMA] * 4
    ),
)(_sc_body)


_ROWS = 512
_STEPS_PER_IMG = _PIX // (128 * _ROWS)
_SLAB = 32

_SQRT2 = np.float32(np.sqrt(2.0))
_LD_CONST = np.float32(0.5 * np.log(2.0 * np.pi) + np.log(_NORM + 1e-8)
                       + np.log(_SIGMA))


def _erfinv(x):
    w = -jnp.log1p(-x * x)
    wc = w - 2.5
    p1 = jnp.float32(2.81022636e-08)
    for c in (3.43273939e-07, -3.5233877e-06, -4.39150654e-06, 0.00021858087,
              -0.00125372503, -0.00417768164, 0.246640727, 1.50140941):
        p1 = p1 * wc + jnp.float32(c)
    wt = jnp.sqrt(w) - 3.0
    p2 = jnp.float32(-0.000200214257)
    for c in (0.000100950558, 0.00134934322, -0.00367342844, 0.00573950773,
              -0.0076224613, 0.00943887047, 1.00167406, 2.83297682):
        p2 = p2 * wt + jnp.float32(c)
    return jnp.where(w < 5.0, p1, p2) * x


def _tc_body(u_ref, q_ref, z_ref, ld_ref):
    j = pl.program_id(0) % _STEPS_PER_IMG

    @pl.when(j == 0)
    def _():
        ld_ref[...] = jnp.zeros((1, 1, 1), jnp.float32)

    def slab(i, acc):
        sl = (0, pl.ds(i * _SLAB, _SLAB), slice(None))
        u = jnp.clip(u_ref[sl], 1e-5, 1.0 - 1e-5)
        z = _erfinv(2.0 * u - 1.0) * _SQRT2
        z_ref[0, 0, pl.ds(i * 8, 8), :] = (
            z.reshape(4, 8, 128).transpose(1, 0, 2).reshape(8, _W))
        return acc + (jnp.log(q_ref[sl]) + 0.5 * (z * z))

    acc = lax.fori_loop(0, _ROWS // _SLAB, slab,
                        jnp.zeros((_SLAB, 128), jnp.float32), unroll=2)
    tot = jnp.sum(acc) + np.float32(_ROWS * 128) * _LD_CONST
    ld_ref[...] = ld_ref[...] + tot.reshape(1, 1, 1)


def _tc_finish(u, q):
    nsteps = _BS * _STEPS_PER_IMG
    return pl.pallas_call(
        _tc_body,
        grid=(nsteps,),
        in_specs=[pl.BlockSpec((1, _ROWS, 128),
                               lambda i: (i // _STEPS_PER_IMG,
                                          i % _STEPS_PER_IMG, 0))] * 2,
        out_specs=[pl.BlockSpec((1, 1, _H // _STEPS_PER_IMG, _W),
                                lambda i: (i // _STEPS_PER_IMG, 0,
                                           i % _STEPS_PER_IMG, 0)),
                   pl.BlockSpec((1, 1, 1),
                                lambda i: (i // _STEPS_PER_IMG, 0, 0))],
        out_shape=[jax.ShapeDtypeStruct((_BS, 1, _H, _W), jnp.float32),
                   jax.ShapeDtypeStruct((_BS, 1, 1), jnp.float32)],
    )(u, q)


def kernel(x, clean, x_grid, pdf_table, cdf_table):
    denom = (x_grid[1:] - x_grid[:-1]) + 1e-8
    b_c = (cdf_table[1:] - cdf_table[:-1]) / denom
    a_c = cdf_table[:-1] - b_c * x_grid[:-1]
    b_p = (pdf_table[1:] - pdf_table[:-1]) / denom
    a_p = pdf_table[:-1] - b_p * x_grid[:-1]
    pad = jnp.zeros((1,), jnp.float32)
    a_c = jnp.concatenate([pad, a_c])
    b_c = jnp.concatenate([pad, b_c])
    a_p = jnp.concatenate([pad, a_p])
    b_p = jnp.concatenate([pad, b_p])

    s3 = (_BS, _PIX // 128, 128)
    u, q = _sc_interp(x, clean, a_c, b_c, a_p, b_p)
    z2, ld = _tc_finish(u.reshape(s3), q.reshape(s3))
    return z2, ld.reshape(_B)


# speedup vs baseline: 1.7006x; 1.0166x over previous
import functools

import jax
import jax.numpy as jnp
import numpy as np
from jax import lax
from jax.experimental import pallas as pl
from jax.experimental.pallas import tpu as pltpu
from jax.experimental.pallas import tpu_sc as plsc

_BIAS = 500.0
_SIGMA = 20.0
_GAIN = 300.0
_NORM = 2000.0
_NBINS = 30000
_XG0 = 380.0
_XGL = 65535.0
_INV_DX = np.float32((_NBINS - 1) / (_XGL - _XG0))

_NC, _NS = 2, 16
_NW = _NC * _NS
_B, _H, _W = 16, 512, 512
_TOTAL = _B * _H * _W
_PIX = _H * _W
_CH = 1024

_NSPLIT = 1
_BS = _B // _NSPLIT
_TOT_S = _TOTAL // _NSPLIT
_PER_W = _TOT_S // _NW
_NPAIR = _PER_W // (2 * _CH)


def _sc_body(x_hbm, cl_hbm, ac_hbm, bc_hbm, ap_hbm, bp_hbm,
             u_hbm, p_hbm, ac_v, bc_v, ap_v, bp_v,
             xb0, cb0, ub0, pb0, xb1, cb1, ub1, pb1,
             sin0, sin1, sout0, sout1):
    wid = lax.axis_index("s") * _NC + lax.axis_index("c")
    base = wid * _PER_W
    tdesc = [pltpu.async_copy(src, dst, sin0) for src, dst in
             ((ac_hbm, ac_v), (bc_hbm, bc_v), (ap_hbm, ap_v), (bp_hbm, bp_v))]
    for d in tdesc:
        d.wait()

    def issue_in(ci, xb, cb, sem):
        tci = base // _CH + ci
        img = lax.shift_right_logical(tci, 8)
        rem = lax.bitwise_and(tci, 255)
        t = lax.shift_right_logical(rem, 2)
        c = lax.bitwise_and(rem, 3)
        tile = (img, 0, pl.ds(8 * t, 8), pl.ds(128 * c, 128))
        pltpu.async_copy(x_hbm.at[tile], xb, sem)
        pltpu.async_copy(cl_hbm.at[tile], cb, sem)

    def drain_in(xb, cb, sem):
        tile0 = (0, 0, pl.ds(0, 8), pl.ds(0, 128))
        pltpu.make_async_copy(x_hbm.at[tile0], xb, sem).wait()
        pltpu.make_async_copy(cl_hbm.at[tile0], cb, sem).wait()

    def issue_out(ci, ub, pb, sem):
        off = base + ci * _CH
        pltpu.async_copy(ub, u_hbm.at[pl.ds(off, _CH)], sem)
        pltpu.async_copy(pb, p_hbm.at[pl.ds(off, _CH)], sem)

    def drain_out(ub, pb, sem):
        pltpu.make_async_copy(ub, u_hbm.at[pl.ds(base, _CH)], sem).wait()
        pltpu.make_async_copy(pb, p_hbm.at[pl.ds(base, _CH)], sem).wait()

    def compute(xb, cb, ub, pb):
        @plsc.parallel_loop(0, _CH // 16, step=1, unroll=8)
        def _(e):
            row = lax.shift_right_logical(e, 3)
            col = 16 * lax.bitwise_and(e, 7)
            sl = pl.ds(16 * e, 16)
            xv = xb[row, pl.ds(col, 16)]
            cv = cb[row, pl.ds(col, 16)]
            sig = jnp.maximum(cv * _NORM - _BIAS, 0.0)
            var = (2.0 * _GAIN) * sig + (_SIGMA * _SIGMA)
            r = lax.bitcast_convert_type(
                jnp.int32(0x5F3759DF) - lax.shift_right_arithmetic(
                    lax.bitcast_convert_type(var, jnp.int32), 1), jnp.float32)
            h = 0.5 * var
            r = r * (1.5 - h * r * r)
            r = r * (1.5 - h * r * r)
            sf = _SIGMA * r
            xc = (xv * _NORM) * sf + _BIAS
            xc = jnp.minimum(jnp.maximum(xc, _XG0), _XGL)
            posi = ((xc - _XG0) * _INV_DX).astype(jnp.int32)
            idx = jnp.maximum(jnp.minimum(posi + 1, _NBINS - 1), 1)
            ub[sl] = plsc.load_gather(ac_v, [idx]) + plsc.load_gather(bc_v, [idx]) * xc
            p = plsc.load_gather(ap_v, [idx]) + plsc.load_gather(bp_v, [idx]) * xc
            pb[sl] = (p + 1e-8) * r

    issue_in(0, xb0, cb0, sin0)

    def pair(k, _):
        c0 = 2 * k
        issue_in(c0 + 1, xb1, cb1, sin1)
        drain_in(xb0, cb0, sin0)

        @pl.when(k > 0)
        def _():
            drain_out(ub0, pb0, sout0)

        compute(xb0, cb0, ub0, pb0)
        issue_out(c0, ub0, pb0, sout0)

        @pl.when(k < _NPAIR - 1)
        def _():
            issue_in(c0 + 2, xb0, cb0, sin0)

        drain_in(xb1, cb1, sin1)

        @pl.when(k > 0)
        def _():
            drain_out(ub1, pb1, sout1)

        compute(xb1, cb1, ub1, pb1)
        issue_out(c0 + 1, ub1, pb1, sout1)
        return _

    lax.fori_loop(0, _NPAIR, pair, None)
    drain_out(ub0, pb0, sout0)
    drain_out(ub1, pb1, sout1)


_sc_interp = functools.partial(
    pl.kernel,
    out_type=(jax.ShapeDtypeStruct((_TOT_S,), jnp.float32),
              jax.ShapeDtypeStruct((_TOT_S,), jnp.float32)),
    mesh=plsc.VectorSubcoreMesh(core_axis_name="c", subcore_axis_name="s",
                                num_cores=_NC, num_subcores=_NS),
    compiler_params=pltpu.CompilerParams(needs_layout_passes=False,
                                         use_tc_tiling_on_sc=True),
    scratch_types=(
        [pltpu.VMEM((_NBINS,), jnp.float32)] * 4
        + [pltpu.VMEM((8, 128), jnp.float32),
           pltpu.VMEM((8, 128), jnp.float32),
           pltpu.VMEM((_CH,), jnp.float32),
           pltpu.VMEM((_CH,), jnp.float32)] * 2
        + [pltpu.SemaphoreType.DMA] * 4
    ),
)(_sc_body)


_ROWS = 512
_STEPS_PER_IMG = _PIX // (128 * _ROWS)
_SLAB = 32

_SQRT2 = np.float32(np.sqrt(2.0))
_LD_CONST = np.float32(0.5 * np.log(2.0 * np.pi) + np.log(_NORM + 1e-8)
                       + np.log(_SIGMA))


def _erfinv(x):
    w = -jnp.log1p(-x * x)
    wc = w - 2.5
    p1 = jnp.float32(2.81022636e-08)
    for c in (3.43273939e-07, -3.5233877e-06, -4.39150654e-06, 0.00021858087,
              -0.00125372503, -0.00417768164, 0.246640727, 1.50140941):
        p1 = p1 * wc + jnp.float32(c)
    wt = jnp.sqrt(w) - 3.0
    p2 = jnp.float32(-0.000200214257)
    for c in (0.000100950558, 0.00134934322, -0.00367342844, 0.00573950773,
              -0.0076224613, 0.00943887047, 1.00167406, 2.83297682):
        p2 = p2 * wt + jnp.float32(c)
    return jnp.where(w < 5.0, p1, p2) * x


def _tc_body(u_ref, q_ref, z_ref, ld_ref):
    j = pl.program_id(0) % _STEPS_PER_IMG

    @pl.when(j == 0)
    def _():
        ld_ref[...] = jnp.zeros((1, 1, 1), jnp.float32)

    def slab(i, acc):
        sl = (0, pl.ds(i * _SLAB, _SLAB), slice(None))
        u = jnp.clip(u_ref[sl], 1e-5, 1.0 - 1e-5)
        z = _erfinv(2.0 * u - 1.0) * _SQRT2
        z_ref[0, 0, pl.ds(i * 8, 8), :] = (
            z.reshape(4, 8, 128).transpose(1, 0, 2).reshape(8, _W))
        return acc + (jnp.log(q_ref[sl]) + 0.5 * (z * z))

    acc = lax.fori_loop(0, _ROWS // _SLAB, slab,
                        jnp.zeros((_SLAB, 128), jnp.float32), unroll=4)
    tot = jnp.sum(acc) + np.float32(_ROWS * 128) * _LD_CONST
    ld_ref[...] = ld_ref[...] + tot.reshape(1, 1, 1)


def _tc_finish(u, q):
    nsteps = _BS * _STEPS_PER_IMG
    return pl.pallas_call(
        _tc_body,
        grid=(nsteps,),
        in_specs=[pl.BlockSpec((1, _ROWS, 128),
                               lambda i: (i // _STEPS_PER_IMG,
                                          i % _STEPS_PER_IMG, 0))] * 2,
        out_specs=[pl.BlockSpec((1, 1, _H // _STEPS_PER_IMG, _W),
                                lambda i: (i // _STEPS_PER_IMG, 0,
                                           i % _STEPS_PER_IMG, 0)),
                   pl.BlockSpec((1, 1, 1),
                                lambda i: (i // _STEPS_PER_IMG, 0, 0))],
        out_shape=[jax.ShapeDtypeStruct((_BS, 1, _H, _W), jnp.float32),
                   jax.ShapeDtypeStruct((_BS, 1, 1), jnp.float32)],
    )(u, q)


def kernel(x, clean, x_grid, pdf_table, cdf_table):
    denom = (x_grid[1:] - x_grid[:-1]) + 1e-8
    b_c = (cdf_table[1:] - cdf_table[:-1]) / denom
    a_c = cdf_table[:-1] - b_c * x_grid[:-1]
    b_p = (pdf_table[1:] - pdf_table[:-1]) / denom
    a_p = pdf_table[:-1] - b_p * x_grid[:-1]
    pad = jnp.zeros((1,), jnp.float32)
    a_c = jnp.concatenate([pad, a_c])
    b_c = jnp.concatenate([pad, b_c])
    a_p = jnp.concatenate([pad, a_p])
    b_p = jnp.concatenate([pad, b_p])

    s3 = (_BS, _PIX // 128, 128)
    u, q = _sc_interp(x, clean, a_c, b_c, a_p, b_p)
    z2, ld = _tc_finish(u.reshape(s3), q.reshape(s3))
    return z2, ld.reshape(_B)
